# baseline (speedup 1.0000x reference)
import functools

import jax
import jax.numpy as jnp
from jax import lax
from jax.experimental import pallas as pl
from jax.experimental.pallas import tpu as pltpu
from jax.experimental.pallas import tpu_sc as plsc

T = 26
B = 1024
L = 20
E = 100000
D = 64

NC = 2
NS = 16
NW = NC * NS
CHUNK = 32
NPAIR = T // 2
NBLK = B // CHUNK
NSUPER = NPAIR * NBLK
SPW = NSUPER // NW
HPW = 2 * SPW
ROWS = CHUNK * L
GSZ = 128
NG = ROWS // GSZ
LANES = 16
W2 = T * E // 2


def kernel(indices, offsets, weights):
    del offsets

    pos = jnp.arange(T * B * L, dtype=jnp.int32)
    trow = (pos // (B * L)) * E + indices
    pidx = trow >> 1
    hbit = jnp.pad((trow & 1).reshape(T * B, L),
                   ((0, 0), (0, 32 - L))).reshape(-1)
    w2 = weights.reshape(W2, 2 * D)

    mesh = plsc.VectorSubcoreMesh(core_axis_name="c", subcore_axis_name="s")

    @functools.partial(
        pl.kernel,
        out_type=jax.ShapeDtypeStruct((B, T * D), jnp.float32),
        mesh=mesh,
        scratch_types=[
            pltpu.VMEM((2, ROWS), jnp.int32),
            pltpu.VMEM((2, CHUNK * 32), jnp.int32),
            pltpu.VMEM((ROWS, 2 * D), jnp.float32),
            pltpu.VMEM((2, CHUNK, 2 * D), jnp.float32),
            pltpu.SemaphoreType.DMA((2,)),
            pltpu.SemaphoreType.DMA((2,)),
            pltpu.SemaphoreType.DMA,
            pltpu.SemaphoreType.DMA((2,)),
        ],
    )
    def tbe(pidx_hbm, hbit_hbm, w_hbm, out_hbm,
            idx_v, hb_v, rows_v, out_v, isem, hsem, gsem, osem):
        wid = lax.axis_index("s") * NC + lax.axis_index("c")
        s0 = wid * SPW

        def coords(j):
            m = j // 2
            s = s0 + m
            p = s // NBLK
            b0 = (s % NBLK) * CHUNK
            return m, p, b0

        def in_off(j):
            m, p, b0 = coords(j)
            t = 2 * p + (j % 2)
            return t * (B * L) + b0 * L

        def idx_copies(j, buf):
            o = in_off(j)
            m, p, b0 = coords(j)
            t = 2 * p + (j % 2)
            ho = (t * B + b0) * 32
            return (
                pltpu.make_async_copy(
                    pidx_hbm.at[pl.ds(o, ROWS)], idx_v.at[buf], isem.at[buf]),
                pltpu.make_async_copy(
                    hbit_hbm.at[pl.ds(ho, CHUNK * 32)], hb_v.at[buf],
                    hsem.at[buf]),
            )

        def gather_copies(buf):
            return [
                pltpu.make_async_copy(
                    w_hbm.at[idx_v.at[buf].at[pl.ds(k * GSZ, GSZ)]],
                    rows_v.at[pl.ds(k * GSZ, GSZ)],
                    gsem)
                for k in range(NG)
            ]

        def out_copy(j):
            m, p, b0 = coords(j)
            ob = m % 2
            return pltpu.make_async_copy(
                out_v.at[ob],
                out_hbm.at[pl.ds(b0, CHUNK), pl.ds(p * 2 * D, 2 * D)],
                osem.at[ob])

        def accumulate(j, buf, h):
            m = j // 2
            ob = m % 2

            @pl.loop(0, CHUNK)
            def bag_loop(bg):
                r0 = bg * L
                hbv0 = hb_v[buf, pl.ds(bg * 32, LANES)]
                hbv1 = hb_v[buf, pl.ds(bg * 32 + LANES, LANES)]
                hbs = [hbv0[el] for el in range(LANES)]
                hbs += [hbv1[el] for el in range(L - LANES)]
                for d in range(D // LANES):
                    src = pl.ds(d * LANES, LANES)
                    alt = pl.ds(D + d * LANES, LANES)
                    acc = None
                    for el in range(L):
                        lo = rows_v[r0 + el, src]
                        hi = rows_v[r0 + el, alt]
                        v = jnp.where(hbs[el] > 0, hi, lo)
                        acc = v if acc is None else acc + v
                    out_v[ob, bg, pl.ds(h * D + d * LANES, LANES)] = acc

        for cp in idx_copies(0, 0):
            cp.start()
        for cp in idx_copies(0, 0):
            cp.wait()
        for cp in gather_copies(0):
            cp.start()
        for cp in idx_copies(1, 1):
            cp.start()

        def halfbody(j, buf):
            h = buf
            nbuf = 1 - buf

            for cp in gather_copies(buf):
                cp.wait()

            if h == 0:
                @pl.when(j >= 4)
                def _():
                    out_copy(j - 4).wait()

            accumulate(j, buf, h)

            if h == 1:
                out_copy(j).start()

            @pl.when(j + 2 < HPW)
            def _():
                for cp in idx_copies(j + 2, buf):
                    cp.start()

            @pl.when(j + 1 < HPW)
            def _():
                for cp in idx_copies(j + 1, nbuf):
                    cp.wait()
                for cp in gather_copies(nbuf):
                    cp.start()

        @pl.loop(0, SPW)
        def super_loop(m):
            j0 = m * 2
            halfbody(j0, 0)
            halfbody(j0 + 1, 1)

        out_copy(HPW - 4).wait()
        out_copy(HPW - 2).wait()

    return tbe(pidx, hbit, w2)


# speedup vs baseline: 1.0292x; 1.0292x over previous
import functools

import jax
import jax.numpy as jnp
from jax import lax
from jax.experimental import pallas as pl
from jax.experimental.pallas import tpu as pltpu
from jax.experimental.pallas import tpu_sc as plsc

T = 26
B = 1024
L = 20
E = 100000
D = 64

NC = 2
NS = 16
NW = NC * NS
CHUNK = 32
NPAIR = T // 2
NBLK = B // CHUNK
NSUPER = NPAIR * NBLK
SPW = NSUPER // NW
HPW = 2 * SPW
ROWS = CHUNK * L
NSUB = 4
SROWS = ROWS // NSUB
SBAGS = CHUNK // NSUB
LANES = 16
W2 = T * E // 2


def kernel(indices, offsets, weights):
    del offsets

    pos = jnp.arange(T * B * L, dtype=jnp.int32)
    trow = (pos // (B * L)) * E + indices
    pidx = jnp.pad((trow >> 1).reshape(T * B * NSUB // CHUNK, SBAGS * L),
                   ((0, 0), (0, 256 - SBAGS * L))).reshape(-1)
    hbit = jnp.pad((trow & 1).reshape(T * B, L),
                   ((0, 0), (0, 32 - L))).reshape(-1)
    w2 = weights.reshape(W2, 2 * D)

    mesh = plsc.VectorSubcoreMesh(core_axis_name="c", subcore_axis_name="s")

    @functools.partial(
        pl.kernel,
        out_type=jax.ShapeDtypeStruct((B, T * D), jnp.float32),
        mesh=mesh,
        scratch_types=[
            pltpu.VMEM((2, NSUB * 256), jnp.int32),
            pltpu.VMEM((2, CHUNK * 32), jnp.int32),
            pltpu.VMEM((2, SROWS, 2 * D), jnp.float32),
            pltpu.VMEM((2, CHUNK, 2 * D), jnp.float32),
            pltpu.SemaphoreType.DMA((2,)),
            pltpu.SemaphoreType.DMA((2,)),
            pltpu.SemaphoreType.DMA((2,)),
            pltpu.SemaphoreType.DMA((2,)),
        ],
    )
    def tbe(pidx_hbm, hbit_hbm, w_hbm, out_hbm,
            idx_v, hb_v, rows_v, out_v, isem, hsem, gsem, osem):
        wid = lax.axis_index("s") * NC + lax.axis_index("c")
        s0 = wid * SPW

        def coords(j):
            m = j // 2
            s = s0 + m
            p = s // NBLK
            b0 = (s % NBLK) * CHUNK
            return m, p, b0

        def idx_copies(j, buf):
            m, p, b0 = coords(j)
            t = 2 * p + (j % 2)
            o = (t * B + b0) // SBAGS * 256
            ho = (t * B + b0) * 32
            return (
                pltpu.make_async_copy(
                    pidx_hbm.at[pl.ds(o, NSUB * 256)], idx_v.at[buf],
                    isem.at[buf]),
                pltpu.make_async_copy(
                    hbit_hbm.at[pl.ds(ho, CHUNK * 32)], hb_v.at[buf],
                    hsem.at[buf]),
            )

        def gather_copies(jbuf, s, gbuf):
            iv = idx_v.at[jbuf]
            return [
                pltpu.make_async_copy(
                    w_hbm.at[iv.at[pl.ds(s * 256, 128)]],
                    rows_v.at[gbuf].at[pl.ds(0, 128)],
                    gsem.at[gbuf]),
                pltpu.make_async_copy(
                    w_hbm.at[iv.at[pl.ds(s * 256 + 128, SROWS - 128)]],
                    rows_v.at[gbuf].at[pl.ds(128, SROWS - 128)],
                    gsem.at[gbuf]),
            ]

        def out_copy(j):
            m, p, b0 = coords(j)
            ob = m % 2
            return pltpu.make_async_copy(
                out_v.at[ob],
                out_hbm.at[pl.ds(b0, CHUNK), pl.ds(p * 2 * D, 2 * D)],
                osem.at[ob])

        def accumulate_sub(j, jbuf, h, s, gbuf):
            m = j // 2
            ob = m % 2

            @pl.loop(0, SBAGS)
            def bag_loop(i):
                bg = s * SBAGS + i
                r0 = i * L
                hbv0 = hb_v[jbuf, pl.ds(bg * 32, LANES)]
                hbv1 = hb_v[jbuf, pl.ds(bg * 32 + LANES, LANES)]
                hbs = [hbv0[el] for el in range(LANES)]
                hbs += [hbv1[el] for el in range(L - LANES)]
                accs = [None] * (D // LANES)
                for el in range(L):
                    off = hbs[el] * D
                    for d in range(D // LANES):
                        start = pl.multiple_of(off + d * LANES, LANES)
                        v = rows_v[gbuf, r0 + el, pl.ds(start, LANES)]
                        accs[d] = v if accs[d] is None else accs[d] + v
                for d in range(D // LANES):
                    out_v[ob, bg, pl.ds(h * D + d * LANES, LANES)] = accs[d]

        for cp in idx_copies(0, 0):
            cp.start()
        for cp in idx_copies(0, 0):
            cp.wait()
        for cp in gather_copies(0, 0, 0):
            cp.start()
        for cp in idx_copies(1, 1):
            cp.start()

        def halfbody(j, jbuf):
            h = jbuf
            nbuf = 1 - jbuf

            if h == 0:
                @pl.when(j >= 4)
                def _():
                    out_copy(j - 4).wait()

            for s in range(NSUB):
                gbuf = s % 2
                for cp in gather_copies(jbuf, s, gbuf):
                    cp.wait()

                if s < NSUB - 1:
                    for cp in gather_copies(jbuf, s + 1, 1 - gbuf):
                        cp.start()
                else:
                    @pl.when(j + 1 < HPW)
                    def _():
                        for cp in idx_copies(j + 1, nbuf):
                            cp.wait()
                        for cp in gather_copies(nbuf, 0, 1 - gbuf):
                            cp.start()

                accumulate_sub(j, jbuf, h, s, gbuf)

            if h == 1:
                out_copy(j).start()

            @pl.when(j + 2 < HPW)
            def _():
                for cp in idx_copies(j + 2, jbuf):
                    cp.start()

        @pl.loop(0, SPW)
        def super_loop(m):
            j0 = m * 2
            halfbody(j0, 0)
            halfbody(j0 + 1, 1)

        out_copy(HPW - 4).wait()
        out_copy(HPW - 2).wait()

    return tbe(pidx, hbit, w2)
